# baseline (speedup 1.0000x reference)
import functools

import jax
import jax.numpy as jnp
from jax import lax
from jax.experimental import pallas as pl
from jax.experimental.pallas import tpu as pltpu
from jax.experimental.pallas import tpu_sc as plsc

N = 10000
E = 320000
H = 64

_BE = 6400
_BN = 2000

_NC = 2
_NS = 16
_NW = _NC * _NS
_GC = 200
_SCC = 1000
_E2 = E // 2
_PW = _E2 // _NW
_GNCH = _PW // _GC
_STRIPE = N // _NS


def _gather_sc(p12, src, dst):
    mesh = plsc.VectorSubcoreMesh(core_axis_name="c", subcore_axis_name="s")
    f32 = jnp.float32

    @functools.partial(
        pl.kernel, mesh=mesh,
        out_type=jax.ShapeDtypeStruct((_E2, H), f32),
        scratch_types=[pltpu.VMEM((_GC,), jnp.int32),
                       pltpu.VMEM((_GC,), jnp.int32),
                       pltpu.VMEM((_GC,), jnp.int32),
                       pltpu.VMEM((_GC,), jnp.int32),
                       pltpu.VMEM((_GC, 2 * H), f32),
                       pltpu.VMEM((_GC, 2 * H), f32),
                       pltpu.VMEM((_GC, 2 * H), f32),
                       pltpu.VMEM((_GC, 2 * H), f32),
                       pltpu.VMEM((_GC, H), f32),
                       pltpu.SemaphoreType.DMA,
                       pltpu.SemaphoreType.DMA,
                       pltpu.SemaphoreType.DMA,
                       pltpu.SemaphoreType.DMA],
    )
    def k(p12_hbm, src_hbm, dst_hbm, g_hbm,
          i1a, i2a, i1b, i2b, r1a, r2a, r1b, r2b, o_v,
          s1a, s2a, s1b, s2b):
        wid = lax.axis_index("s") * _NC + lax.axis_index("c")
        base = wid * _PW

        def load_issue(c, i1, i2, r1, r2, s1, s2):
            off = base + c * _GC
            pltpu.sync_copy(src_hbm.at[pl.ds(off, _GC)], i1)
            pltpu.sync_copy(dst_hbm.at[pl.ds(off, _GC)], i2)
            pltpu.async_copy(p12_hbm.at[i1], r1, s1)
            pltpu.async_copy(p12_hbm.at[i2], r2, s2)

        def wait(i1, i2, r1, r2, s1, s2):
            pltpu.make_async_copy(p12_hbm.at[i1], r1, s1).wait()
            pltpu.make_async_copy(p12_hbm.at[i2], r2, s2).wait()

        def add_wb(c, r1, r2):
            def rowgrp(j, carry):
                for q in range(4):
                    r = j * 4 + q
                    for kk in range(4):
                        lo = pl.ds(kk * 16, 16)
                        hi = pl.ds(H + kk * 16, 16)
                        o_v[r, lo] = r1[r, lo] + r2[r, hi]
                return carry
            lax.fori_loop(0, _GC // 4, rowgrp, 0)
            pltpu.sync_copy(o_v, g_hbm.at[pl.ds(base + c * _GC, _GC)])

        load_issue(0, i1a, i2a, r1a, r2a, s1a, s2a)

        def body(j, carry):
            ca = 2 * j
            cb = 2 * j + 1
            wait(i1a, i2a, r1a, r2a, s1a, s2a)
            load_issue(cb, i1b, i2b, r1b, r2b, s1b, s2b)
            add_wb(ca, r1a, r2a)
            wait(i1b, i2b, r1b, r2b, s1b, s2b)
            load_issue(lax.rem(cb + 1, _GNCH), i1a, i2a, r1a, r2a, s1a, s2a)
            add_wb(cb, r1b, r2b)
            return carry

        lax.fori_loop(0, _GNCH // 2, body, 0)
        wait(i1a, i2a, r1a, r2a, s1a, s2a)
        if _GNCH % 2 == 1:
            add_wb(_GNCH - 1, r1a, r2a)

    return k(p12, src, dst)


def _scatter_sc(he, dst, zeros):
    mesh = plsc.VectorSubcoreMesh(core_axis_name="c", subcore_axis_name="s")
    f32 = jnp.float32

    @functools.partial(
        pl.kernel, mesh=mesh,
        compiler_params=pltpu.CompilerParams(use_tc_tiling_on_sc=False),
        out_type=jax.ShapeDtypeStruct((_NC, N, H), f32),
        scratch_types=[pltpu.VMEM((_SCC,), jnp.int32),
                       pltpu.VMEM((_SCC, H), f32),
                       pltpu.VMEM_SHARED((N, H), f32),
                       pltpu.SemaphoreType.DMA],
    )
    def k(he_hbm, dst_hbm, z_hbm, out_hbm, idx_v, rows_v, acc_sh, sem):
        cid = lax.axis_index("c")
        sid = lax.axis_index("s")
        wid = sid * _NC + cid
        r0 = sid * _STRIPE
        pltpu.sync_copy(z_hbm.at[pl.ds(r0, _STRIPE)],
                        acc_sh.at[pl.ds(r0, _STRIPE)])
        plsc.subcore_barrier()
        base = wid * _PW

        def body(i, carry):
            off = base + i * _SCC
            pltpu.sync_copy(dst_hbm.at[pl.ds(off, _SCC)], idx_v)
            pltpu.sync_copy(he_hbm.at[pl.ds(off, _SCC)], rows_v)
            pltpu.sync_copy(rows_v, acc_sh.at[idx_v], add=True)
            return carry

        lax.fori_loop(0, _PW // _SCC, body, 0)
        plsc.subcore_barrier()
        pltpu.sync_copy(acc_sh.at[pl.ds(r0, _STRIPE)],
                        out_hbm.at[cid, pl.ds(r0, _STRIPE)])

    return k(he, dst, zeros)


def _gru_edge(g, he, W3Whh, mb, Wih, bih, bhh):
    hw = he @ W3Whh
    m = jnp.maximum(g + hw[:, :H] + mb, 0.0)
    gi = m @ Wih + bih
    gh = hw[:, H:] + bhh
    r = jax.nn.sigmoid(gi[:, :H] + gh[:, :H])
    z = jax.nn.sigmoid(gi[:, H:2 * H] + gh[:, H:2 * H])
    n = jnp.tanh(gi[:, 2 * H:] + r * gh[:, 2 * H:])
    return (1.0 - z) * n + z * he


def _edge_body_first(g, efT, eW, eb, W3Whh, mb, Wih, bih, bhh, he_out):
    he = lax.dot_general(efT[...], eW[...], (((0,), (0,)), ((), ()))) + eb[...]
    he_out[...] = _gru_edge(g[...], he, W3Whh[...], mb[...],
                            Wih[...], bih[...], bhh[...])


def _edge_body_mid(g, he_in, W3Whh, mb, Wih, bih, bhh, he_out):
    he_out[...] = _gru_edge(g[...], he_in[...], W3Whh[...], mb[...],
                            Wih[...], bih[...], bhh[...])


def _edge_body_last(g, he_in, W3Whh, mb, Wih, bih, bhh, dW, db,
                    he_out, uef_out):
    he = _gru_edge(g[...], he_in[...], W3Whh[...], mb[...],
                   Wih[...], bih[...], bhh[...])
    he_out[...] = he
    uef_out[...] = he @ dW[...] + db[...]


def _eb(shape):
    return pl.BlockSpec(shape, lambda i: (i, 0))


def _wb(shape):
    return pl.BlockSpec(shape, lambda i: (0, 0))


def _edge_call(variant, args):
    grid = (_E2 // _BE,)
    f32 = jnp.float32
    if variant == 0:
        in_specs = [_eb((_BE, H)),
                    pl.BlockSpec((16, _BE), lambda i: (0, i)),
                    _wb((16, H)), _wb((1, H)), _wb((H, 4 * H)), _wb((1, H)),
                    _wb((H, 3 * H)), _wb((1, 3 * H)), _wb((1, 3 * H))]
        out_specs = _eb((_BE, H))
        out_shape = jax.ShapeDtypeStruct((_E2, H), f32)
        body = _edge_body_first
    elif variant == 1:
        in_specs = [_eb((_BE, H)), _eb((_BE, H)),
                    _wb((H, 4 * H)), _wb((1, H)),
                    _wb((H, 3 * H)), _wb((1, 3 * H)), _wb((1, 3 * H))]
        out_specs = _eb((_BE, H))
        out_shape = jax.ShapeDtypeStruct((_E2, H), f32)
        body = _edge_body_mid
    else:
        in_specs = [_eb((_BE, H)), _eb((_BE, H)),
                    _wb((H, 4 * H)), _wb((1, H)),
                    _wb((H, 3 * H)), _wb((1, 3 * H)), _wb((1, 3 * H)),
                    _wb((H, 16)), _wb((1, 16))]
        out_specs = [_eb((_BE, H)), _eb((_BE, 16))]
        out_shape = [jax.ShapeDtypeStruct((_E2, H), f32),
                     jax.ShapeDtypeStruct((_E2, 16), f32)]
        body = _edge_body_last
    return pl.pallas_call(body, grid=grid, in_specs=in_specs,
                          out_specs=out_specs, out_shape=out_shape)(*args)


def _prep_body(nf, encW, encb, W12, hn_out, p12_out):
    hn = nf[...] @ encW[...] + encb[...]
    hn_out[...] = hn
    p12_out[...] = hn @ W12[...]


def _prep_call(nf, encW, encb, W12):
    f32 = jnp.float32
    return pl.pallas_call(
        _prep_body, grid=(N // _BN,),
        in_specs=[_eb((_BN, 128)), _wb((128, H)), _wb((1, H)), _wb((H, 2 * H))],
        out_specs=[_eb((_BN, H)), _eb((_BN, 2 * H))],
        out_shape=[jax.ShapeDtypeStruct((N, H), f32),
                   jax.ShapeDtypeStruct((N, 2 * H), f32)],
    )(nf, encW, encb, W12)


def _gru_node(agg, hn, Wih, bih, Whh, bhh):
    gi = agg @ Wih + bih
    gh = hn @ Whh + bhh
    r = jax.nn.sigmoid(gi[:, :H] + gh[:, :H])
    z = jax.nn.sigmoid(gi[:, H:2 * H] + gh[:, H:2 * H])
    n = jnp.tanh(gi[:, 2 * H:] + r * gh[:, 2 * H:])
    return (1.0 - z) * n + z * hn


def _node_body_mid(aggpA, aggpB, hn_in, Wih, bih, Whh, bhh, W12,
                   hn_out, p12_out):
    agg = jnp.sum(aggpA[...], axis=0) + jnp.sum(aggpB[...], axis=0)
    hn = _gru_node(agg, hn_in[...], Wih[...], bih[...], Whh[...], bhh[...])
    hn_out[...] = hn
    p12_out[...] = hn @ W12[...]


def _node_body_last(aggpA, aggpB, hn_in, Wih, bih, Whh, bhh, dW, db, unf_out):
    agg = jnp.sum(aggpA[...], axis=0) + jnp.sum(aggpB[...], axis=0)
    hn = _gru_node(agg, hn_in[...], Wih[...], bih[...], Whh[...], bhh[...])
    unf_out[...] = hn @ dW[...] + db[...]


def _node_call(variant, aggpA, aggpB, args):
    f32 = jnp.float32
    aspec = pl.BlockSpec((_NC, _BN, H), lambda i: (0, i, 0))
    if variant == 0:
        in_specs = [aspec, aspec, _eb((_BN, H)),
                    _wb((H, 3 * H)), _wb((1, 3 * H)), _wb((H, 3 * H)),
                    _wb((1, 3 * H)), _wb((H, 2 * H))]
        out_specs = [_eb((_BN, H)), _eb((_BN, 2 * H))]
        out_shape = [jax.ShapeDtypeStruct((N, H), f32),
                     jax.ShapeDtypeStruct((N, 2 * H), f32)]
        body = _node_body_mid
    else:
        in_specs = [aspec, aspec, _eb((_BN, H)),
                    _wb((H, 3 * H)), _wb((1, 3 * H)), _wb((H, 3 * H)),
                    _wb((1, 3 * H)), _wb((H, 128)), _wb((1, 128))]
        out_specs = _eb((_BN, 128))
        out_shape = jax.ShapeDtypeStruct((N, 128), f32)
        body = _node_body_last
    return pl.pallas_call(body, grid=(N // _BN,), in_specs=in_specs,
                          out_specs=out_specs, out_shape=out_shape)(aggpA, aggpB,
                                                                    *args)


def kernel(nf, ef, edge_index, node_enc_W, node_enc_b, edge_enc_W, edge_enc_b,
           msg_W, msg_b, e_gru_Wih, e_gru_Whh, e_gru_bih, e_gru_bhh,
           n_gru_Wih, n_gru_Whh, n_gru_bih, n_gru_bhh,
           node_dec_W, node_dec_b, edge_dec_W, edge_dec_b):
    src = edge_index[0]
    dst = edge_index[1]
    W12 = jnp.concatenate([msg_W[:H], msg_W[H:2 * H]], axis=1)
    W3Whh = jnp.concatenate([msg_W[2 * H:], e_gru_Whh], axis=1)
    mb = msg_b.reshape(1, H)
    ebih = e_gru_bih.reshape(1, 3 * H)
    ebhh = e_gru_bhh.reshape(1, 3 * H)
    nbih = n_gru_bih.reshape(1, 3 * H)
    nbhh = n_gru_bhh.reshape(1, 3 * H)

    hn, p12 = _prep_call(nf, node_enc_W, node_enc_b.reshape(1, H), W12)
    zeros = jnp.zeros((N, H), jnp.float32)

    srcs = (src[:_E2], src[_E2:])
    dsts = (dst[:_E2], dst[_E2:])
    efTs = (ef[:_E2].T, ef[_E2:].T)
    ebias = edge_enc_b.reshape(1, H)
    dbias = edge_dec_b.reshape(1, 16)

    he = [None, None]
    uef = [None, None]
    unf = None
    for it in range(3):
        aggp = [None, None]
        for hf in range(2):
            g = _gather_sc(p12, srcs[hf], dsts[hf])
            if it == 0:
                he[hf] = _edge_call(0, (g, efTs[hf], edge_enc_W, ebias,
                                        W3Whh, mb, e_gru_Wih, ebih, ebhh))
            elif it == 1:
                he[hf] = _edge_call(1, (g, he[hf], W3Whh, mb, e_gru_Wih,
                                        ebih, ebhh))
            else:
                he[hf], uef[hf] = _edge_call(2, (g, he[hf], W3Whh, mb,
                                                 e_gru_Wih, ebih, ebhh,
                                                 edge_dec_W, dbias))
            aggp[hf] = _scatter_sc(he[hf], dsts[hf], zeros)
        if it < 2:
            hn, p12 = _node_call(0, aggp[0], aggp[1],
                                 (hn, n_gru_Wih, nbih, n_gru_Whh, nbhh, W12))
        else:
            unf = _node_call(1, aggp[0], aggp[1],
                             (hn, n_gru_Wih, nbih, n_gru_Whh, nbhh,
                              node_dec_W, node_dec_b.reshape(1, 128)))
    return (unf, jnp.concatenate(uef, axis=0))


# speedup vs baseline: 1.0729x; 1.0729x over previous
import functools

import jax
import jax.numpy as jnp
from jax import lax
from jax.experimental import pallas as pl
from jax.experimental.pallas import tpu as pltpu
from jax.experimental.pallas import tpu_sc as plsc

N = 10000
E = 320000
H = 64

_BE = 6400
_BN = 2000

_NC = 2
_NS = 16
_NW = _NC * _NS
_GC = 200
_SCC = 200
_E2 = E // 2
_PW = _E2 // _NW
_GNCH = _PW // _GC
_STRIPE = N // _NS


def _gather_sc(p12, src, dst):
    mesh = plsc.VectorSubcoreMesh(core_axis_name="c", subcore_axis_name="s")
    f32 = jnp.float32

    @functools.partial(
        pl.kernel, mesh=mesh,
        out_type=jax.ShapeDtypeStruct((_E2, H), f32),
        scratch_types=[pltpu.VMEM((_GC,), jnp.int32),
                       pltpu.VMEM((_GC,), jnp.int32),
                       pltpu.VMEM((_GC,), jnp.int32),
                       pltpu.VMEM((_GC,), jnp.int32),
                       pltpu.VMEM((_GC, 2 * H), f32),
                       pltpu.VMEM((_GC, 2 * H), f32),
                       pltpu.VMEM((_GC, 2 * H), f32),
                       pltpu.VMEM((_GC, 2 * H), f32),
                       pltpu.VMEM((_GC, H), f32),
                       pltpu.SemaphoreType.DMA,
                       pltpu.SemaphoreType.DMA,
                       pltpu.SemaphoreType.DMA,
                       pltpu.SemaphoreType.DMA],
    )
    def k(p12_hbm, src_hbm, dst_hbm, g_hbm,
          i1a, i2a, i1b, i2b, r1a, r2a, r1b, r2b, o_v,
          s1a, s2a, s1b, s2b):
        wid = lax.axis_index("s") * _NC + lax.axis_index("c")
        base = wid * _PW

        def load_issue(c, i1, i2, r1, r2, s1, s2):
            off = base + c * _GC
            pltpu.sync_copy(src_hbm.at[pl.ds(off, _GC)], i1)
            pltpu.sync_copy(dst_hbm.at[pl.ds(off, _GC)], i2)
            pltpu.async_copy(p12_hbm.at[i1], r1, s1)
            pltpu.async_copy(p12_hbm.at[i2], r2, s2)

        def wait(i1, i2, r1, r2, s1, s2):
            pltpu.make_async_copy(p12_hbm.at[i1], r1, s1).wait()
            pltpu.make_async_copy(p12_hbm.at[i2], r2, s2).wait()

        def add_wb(c, r1, r2):
            def rowgrp(j, carry):
                for q in range(4):
                    r = j * 4 + q
                    for kk in range(4):
                        lo = pl.ds(kk * 16, 16)
                        hi = pl.ds(H + kk * 16, 16)
                        o_v[r, lo] = r1[r, lo] + r2[r, hi]
                return carry
            lax.fori_loop(0, _GC // 4, rowgrp, 0)
            pltpu.sync_copy(o_v, g_hbm.at[pl.ds(base + c * _GC, _GC)])

        load_issue(0, i1a, i2a, r1a, r2a, s1a, s2a)

        def body(j, carry):
            ca = 2 * j
            cb = 2 * j + 1
            wait(i1a, i2a, r1a, r2a, s1a, s2a)
            load_issue(cb, i1b, i2b, r1b, r2b, s1b, s2b)
            add_wb(ca, r1a, r2a)
            wait(i1b, i2b, r1b, r2b, s1b, s2b)
            load_issue(lax.rem(cb + 1, _GNCH), i1a, i2a, r1a, r2a, s1a, s2a)
            add_wb(cb, r1b, r2b)
            return carry

        lax.fori_loop(0, _GNCH // 2, body, 0)
        wait(i1a, i2a, r1a, r2a, s1a, s2a)
        if _GNCH % 2 == 1:
            add_wb(_GNCH - 1, r1a, r2a)

    return k(p12, src, dst)


def _scatter_sc(he, dst, zeros):
    mesh = plsc.VectorSubcoreMesh(core_axis_name="c", subcore_axis_name="s")
    f32 = jnp.float32

    @functools.partial(
        pl.kernel, mesh=mesh,
        out_type=jax.ShapeDtypeStruct((_NC, N, 2 * H), f32),
        scratch_types=[pltpu.VMEM((_SCC,), jnp.int32),
                       pltpu.VMEM((104, H), f32),
                       pltpu.VMEM((_SCC, 2 * H), f32),
                       pltpu.VMEM_SHARED((N, 2 * H), f32),
                       pltpu.SemaphoreType.DMA],
    )
    def k(he_hbm, dst_hbm, z_hbm, out_hbm, idx_v, hbuf, rows_v, acc_sh, sem):
        cid = lax.axis_index("c")
        sid = lax.axis_index("s")
        wid = sid * _NC + cid
        zv = jnp.zeros((16,), f32)

        def zrow(r, carry):
            for kk in range(4):
                rows_v[r, pl.ds(H + kk * 16, 16)] = zv
            return carry

        lax.fori_loop(0, _SCC, zrow, 0)

        @pl.when(sid < 10)
        def _():
            pltpu.sync_copy(z_hbm.at[pl.ds(sid * 1000, 1000)],
                            acc_sh.at[pl.ds(sid * 1000, 1000)])

        plsc.subcore_barrier()
        base = wid * _PW

        def body(i, carry):
            off = base + i * _SCC
            pltpu.sync_copy(dst_hbm.at[pl.ds(off, _SCC)], idx_v)
            for h0, hl in ((0, 104), (104, 96)):
                pltpu.sync_copy(he_hbm.at[pl.ds(off + h0, hl)],
                                hbuf.at[pl.ds(0, hl)])

                def crow(r, c2):
                    for kk in range(4):
                        sl = pl.ds(kk * 16, 16)
                        rows_v[h0 + r, sl] = hbuf[r, sl]
                    return c2

                lax.fori_loop(0, hl, crow, 0)
            pltpu.sync_copy(rows_v, acc_sh.at[idx_v], add=True)
            return carry

        lax.fori_loop(0, _PW // _SCC, body, 0)
        plsc.subcore_barrier()

        @pl.when(sid < 10)
        def _():
            pltpu.sync_copy(acc_sh.at[pl.ds(sid * 1000, 1000)],
                            out_hbm.at[cid, pl.ds(sid * 1000, 1000)])

    return k(he, dst, zeros)


def _gru_edge(g, he, W3Whh, mb, Wih, bih, bhh):
    hw = he @ W3Whh
    m = jnp.maximum(g + hw[:, :H] + mb, 0.0)
    gi = m @ Wih + bih
    gh = hw[:, H:] + bhh
    r = jax.nn.sigmoid(gi[:, :H] + gh[:, :H])
    z = jax.nn.sigmoid(gi[:, H:2 * H] + gh[:, H:2 * H])
    n = jnp.tanh(gi[:, 2 * H:] + r * gh[:, 2 * H:])
    return (1.0 - z) * n + z * he


def _edge_body_first(g, efT, eW, eb, W3Whh, mb, Wih, bih, bhh, he_out):
    he = lax.dot_general(efT[...], eW[...], (((0,), (0,)), ((), ()))) + eb[...]
    he_out[...] = _gru_edge(g[...], he, W3Whh[...], mb[...],
                            Wih[...], bih[...], bhh[...])


def _edge_body_mid(g, he_in, W3Whh, mb, Wih, bih, bhh, he_out):
    he_out[...] = _gru_edge(g[...], he_in[...], W3Whh[...], mb[...],
                            Wih[...], bih[...], bhh[...])


def _edge_body_last(g, he_in, W3Whh, mb, Wih, bih, bhh, dW, db,
                    he_out, uef_out):
    he = _gru_edge(g[...], he_in[...], W3Whh[...], mb[...],
                   Wih[...], bih[...], bhh[...])
    he_out[...] = he
    uef_out[...] = he @ dW[...] + db[...]


def _eb(shape):
    return pl.BlockSpec(shape, lambda i: (i, 0))


def _wb(shape):
    return pl.BlockSpec(shape, lambda i: (0, 0))


def _edge_call(variant, args):
    grid = (_E2 // _BE,)
    f32 = jnp.float32
    if variant == 0:
        in_specs = [_eb((_BE, H)),
                    pl.BlockSpec((16, _BE), lambda i: (0, i)),
                    _wb((16, H)), _wb((1, H)), _wb((H, 4 * H)), _wb((1, H)),
                    _wb((H, 3 * H)), _wb((1, 3 * H)), _wb((1, 3 * H))]
        out_specs = _eb((_BE, H))
        out_shape = jax.ShapeDtypeStruct((_E2, H), f32)
        body = _edge_body_first
    elif variant == 1:
        in_specs = [_eb((_BE, H)), _eb((_BE, H)),
                    _wb((H, 4 * H)), _wb((1, H)),
                    _wb((H, 3 * H)), _wb((1, 3 * H)), _wb((1, 3 * H))]
        out_specs = _eb((_BE, H))
        out_shape = jax.ShapeDtypeStruct((_E2, H), f32)
        body = _edge_body_mid
    else:
        in_specs = [_eb((_BE, H)), _eb((_BE, H)),
                    _wb((H, 4 * H)), _wb((1, H)),
                    _wb((H, 3 * H)), _wb((1, 3 * H)), _wb((1, 3 * H)),
                    _wb((H, 16)), _wb((1, 16))]
        out_specs = [_eb((_BE, H)), _eb((_BE, 16))]
        out_shape = [jax.ShapeDtypeStruct((_E2, H), f32),
                     jax.ShapeDtypeStruct((_E2, 16), f32)]
        body = _edge_body_last
    return pl.pallas_call(body, grid=grid, in_specs=in_specs,
                          out_specs=out_specs, out_shape=out_shape)(*args)


def _prep_body(nf, encW, encb, W12, hn_out, p12_out):
    hn = nf[...] @ encW[...] + encb[...]
    hn_out[...] = hn
    p12_out[...] = hn @ W12[...]


def _prep_call(nf, encW, encb, W12):
    f32 = jnp.float32
    return pl.pallas_call(
        _prep_body, grid=(N // _BN,),
        in_specs=[_eb((_BN, 128)), _wb((128, H)), _wb((1, H)), _wb((H, 2 * H))],
        out_specs=[_eb((_BN, H)), _eb((_BN, 2 * H))],
        out_shape=[jax.ShapeDtypeStruct((N, H), f32),
                   jax.ShapeDtypeStruct((N, 2 * H), f32)],
    )(nf, encW, encb, W12)


def _gru_node(agg, hn, Wih, bih, Whh, bhh):
    gi = agg @ Wih + bih
    gh = hn @ Whh + bhh
    r = jax.nn.sigmoid(gi[:, :H] + gh[:, :H])
    z = jax.nn.sigmoid(gi[:, H:2 * H] + gh[:, H:2 * H])
    n = jnp.tanh(gi[:, 2 * H:] + r * gh[:, 2 * H:])
    return (1.0 - z) * n + z * hn


def _node_body_mid(aggpA, aggpB, hn_in, Wih, bih, Whh, bhh, W12,
                   hn_out, p12_out):
    agg = (jnp.sum(aggpA[...], axis=0) + jnp.sum(aggpB[...], axis=0))[:, :H]
    hn = _gru_node(agg, hn_in[...], Wih[...], bih[...], Whh[...], bhh[...])
    hn_out[...] = hn
    p12_out[...] = hn @ W12[...]


def _node_body_last(aggpA, aggpB, hn_in, Wih, bih, Whh, bhh, dW, db, unf_out):
    agg = (jnp.sum(aggpA[...], axis=0) + jnp.sum(aggpB[...], axis=0))[:, :H]
    hn = _gru_node(agg, hn_in[...], Wih[...], bih[...], Whh[...], bhh[...])
    unf_out[...] = hn @ dW[...] + db[...]


def _node_call(variant, aggpA, aggpB, args):
    f32 = jnp.float32
    aspec = pl.BlockSpec((_NC, _BN, 2 * H), lambda i: (0, i, 0))
    if variant == 0:
        in_specs = [aspec, aspec, _eb((_BN, H)),
                    _wb((H, 3 * H)), _wb((1, 3 * H)), _wb((H, 3 * H)),
                    _wb((1, 3 * H)), _wb((H, 2 * H))]
        out_specs = [_eb((_BN, H)), _eb((_BN, 2 * H))]
        out_shape = [jax.ShapeDtypeStruct((N, H), f32),
                     jax.ShapeDtypeStruct((N, 2 * H), f32)]
        body = _node_body_mid
    else:
        in_specs = [aspec, aspec, _eb((_BN, H)),
                    _wb((H, 3 * H)), _wb((1, 3 * H)), _wb((H, 3 * H)),
                    _wb((1, 3 * H)), _wb((H, 128)), _wb((1, 128))]
        out_specs = _eb((_BN, 128))
        out_shape = jax.ShapeDtypeStruct((N, 128), f32)
        body = _node_body_last
    return pl.pallas_call(body, grid=(N // _BN,), in_specs=in_specs,
                          out_specs=out_specs, out_shape=out_shape)(aggpA, aggpB,
                                                                    *args)


def kernel(nf, ef, edge_index, node_enc_W, node_enc_b, edge_enc_W, edge_enc_b,
           msg_W, msg_b, e_gru_Wih, e_gru_Whh, e_gru_bih, e_gru_bhh,
           n_gru_Wih, n_gru_Whh, n_gru_bih, n_gru_bhh,
           node_dec_W, node_dec_b, edge_dec_W, edge_dec_b):
    src = edge_index[0]
    dst = edge_index[1]
    W12 = jnp.concatenate([msg_W[:H], msg_W[H:2 * H]], axis=1)
    W3Whh = jnp.concatenate([msg_W[2 * H:], e_gru_Whh], axis=1)
    mb = msg_b.reshape(1, H)
    ebih = e_gru_bih.reshape(1, 3 * H)
    ebhh = e_gru_bhh.reshape(1, 3 * H)
    nbih = n_gru_bih.reshape(1, 3 * H)
    nbhh = n_gru_bhh.reshape(1, 3 * H)

    hn, p12 = _prep_call(nf, node_enc_W, node_enc_b.reshape(1, H), W12)
    zeros = jnp.zeros((N, 2 * H), jnp.float32)

    srcs = (src[:_E2], src[_E2:])
    dsts = (dst[:_E2], dst[_E2:])
    efTs = (ef[:_E2].T, ef[_E2:].T)
    ebias = edge_enc_b.reshape(1, H)
    dbias = edge_dec_b.reshape(1, 16)

    he = [None, None]
    uef = [None, None]
    unf = None
    for it in range(3):
        aggp = [None, None]
        for hf in range(2):
            g = _gather_sc(p12, srcs[hf], dsts[hf])
            if it == 0:
                he[hf] = _edge_call(0, (g, efTs[hf], edge_enc_W, ebias,
                                        W3Whh, mb, e_gru_Wih, ebih, ebhh))
            elif it == 1:
                he[hf] = _edge_call(1, (g, he[hf], W3Whh, mb, e_gru_Wih,
                                        ebih, ebhh))
            else:
                he[hf], uef[hf] = _edge_call(2, (g, he[hf], W3Whh, mb,
                                                 e_gru_Wih, ebih, ebhh,
                                                 edge_dec_W, dbias))
            aggp[hf] = _scatter_sc(he[hf], dsts[hf], zeros)
        if it < 2:
            hn, p12 = _node_call(0, aggp[0], aggp[1],
                                 (hn, n_gru_Wih, nbih, n_gru_Whh, nbhh, W12))
        else:
            unf = _node_call(1, aggp[0], aggp[1],
                             (hn, n_gru_Wih, nbih, n_gru_Whh, nbhh,
                              node_dec_W, node_dec_b.reshape(1, 128)))
    return (unf, jnp.concatenate(uef, axis=0))


# speedup vs baseline: 1.1072x; 1.0320x over previous
import functools

import jax
import jax.numpy as jnp
from jax import lax
from jax.experimental import pallas as pl
from jax.experimental.pallas import tpu as pltpu
from jax.experimental.pallas import tpu_sc as plsc

N = 10000
E = 320000
H = 64

_BE = 6400
_BN = 2000

_NC = 2
_NS = 16
_NW = _NC * _NS
_GC = 200
_SCC = 200
_E2 = E // 2
_PW = _E2 // _NW
_GNCH = _PW // _GC
_STRIPE = N // _NS


def _gather_sc(p12, src, dst):
    mesh = plsc.VectorSubcoreMesh(core_axis_name="c", subcore_axis_name="s")
    f32 = jnp.float32

    @functools.partial(
        pl.kernel, mesh=mesh,
        out_type=jax.ShapeDtypeStruct((_E2, H), f32),
        scratch_types=[pltpu.VMEM((_GC,), jnp.int32),
                       pltpu.VMEM((_GC,), jnp.int32),
                       pltpu.VMEM((_GC,), jnp.int32),
                       pltpu.VMEM((_GC,), jnp.int32),
                       pltpu.VMEM((_GC, 2 * H), f32),
                       pltpu.VMEM((_GC, 2 * H), f32),
                       pltpu.VMEM((_GC, 2 * H), f32),
                       pltpu.VMEM((_GC, 2 * H), f32),
                       pltpu.VMEM((_GC, H), f32),
                       pltpu.SemaphoreType.DMA,
                       pltpu.SemaphoreType.DMA,
                       pltpu.SemaphoreType.DMA,
                       pltpu.SemaphoreType.DMA],
    )
    def k(p12_hbm, src_hbm, dst_hbm, g_hbm,
          i1a, i2a, i1b, i2b, r1a, r2a, r1b, r2b, o_v,
          s1a, s2a, s1b, s2b):
        wid = lax.axis_index("s") * _NC + lax.axis_index("c")
        base = wid * _PW

        def load_issue(c, i1, i2, r1, r2, s1, s2):
            off = base + c * _GC
            pltpu.sync_copy(src_hbm.at[pl.ds(off, _GC)], i1)
            pltpu.sync_copy(dst_hbm.at[pl.ds(off, _GC)], i2)
            pltpu.async_copy(p12_hbm.at[i1], r1, s1)
            pltpu.async_copy(p12_hbm.at[i2], r2, s2)

        def wait(i1, i2, r1, r2, s1, s2):
            pltpu.make_async_copy(p12_hbm.at[i1], r1, s1).wait()
            pltpu.make_async_copy(p12_hbm.at[i2], r2, s2).wait()

        def add_wb(c, r1, r2):
            def rowgrp(j, carry):
                for q in range(4):
                    r = j * 4 + q
                    for kk in range(4):
                        lo = pl.ds(kk * 16, 16)
                        hi = pl.ds(H + kk * 16, 16)
                        o_v[r, lo] = r1[r, lo] + r2[r, hi]
                return carry
            lax.fori_loop(0, _GC // 4, rowgrp, 0)
            pltpu.sync_copy(o_v, g_hbm.at[pl.ds(base + c * _GC, _GC)])

        load_issue(0, i1a, i2a, r1a, r2a, s1a, s2a)

        def body(j, carry):
            ca = 2 * j
            cb = 2 * j + 1
            wait(i1a, i2a, r1a, r2a, s1a, s2a)
            load_issue(cb, i1b, i2b, r1b, r2b, s1b, s2b)
            add_wb(ca, r1a, r2a)
            wait(i1b, i2b, r1b, r2b, s1b, s2b)
            load_issue(lax.rem(cb + 1, _GNCH), i1a, i2a, r1a, r2a, s1a, s2a)
            add_wb(cb, r1b, r2b)
            return carry

        lax.fori_loop(0, _GNCH // 2, body, 0)
        wait(i1a, i2a, r1a, r2a, s1a, s2a)
        if _GNCH % 2 == 1:
            add_wb(_GNCH - 1, r1a, r2a)

    return k(p12, src, dst)


def _scatter_sc(he, dst, zeros):
    mesh = plsc.VectorSubcoreMesh(core_axis_name="c", subcore_axis_name="s")
    f32 = jnp.float32

    @functools.partial(
        pl.kernel, mesh=mesh,
        out_type=jax.ShapeDtypeStruct((_NC, N, 2 * H), f32),
        scratch_types=[pltpu.VMEM((_SCC,), jnp.int32),
                       pltpu.VMEM((104, H), f32),
                       pltpu.VMEM((_SCC, 2 * H), f32),
                       pltpu.VMEM_SHARED((N, 2 * H), f32),
                       pltpu.SemaphoreType.DMA],
    )
    def k(he_hbm, dst_hbm, z_hbm, out_hbm, idx_v, hbuf, rows_v, acc_sh, sem):
        cid = lax.axis_index("c")
        sid = lax.axis_index("s")
        wid = sid * _NC + cid
        zv = jnp.zeros((16,), f32)

        def zrow(r, carry):
            for kk in range(4):
                rows_v[r, pl.ds(H + kk * 16, 16)] = zv
            return carry

        lax.fori_loop(0, _SCC, zrow, 0)

        @pl.when(sid < 10)
        def _():
            pltpu.sync_copy(z_hbm.at[pl.ds(sid * 1000, 1000)],
                            acc_sh.at[pl.ds(sid * 1000, 1000)])

        plsc.subcore_barrier()
        base = wid * _PW

        def body(i, carry):
            off = base + i * _SCC
            pltpu.sync_copy(dst_hbm.at[pl.ds(off, _SCC)], idx_v)
            for h0, hl in ((0, 104), (104, 96)):
                pltpu.sync_copy(he_hbm.at[pl.ds(off + h0, hl)],
                                hbuf.at[pl.ds(0, hl)])

                def crow(r, c2):
                    for kk in range(4):
                        sl = pl.ds(kk * 16, 16)
                        rows_v[h0 + r, sl] = hbuf[r, sl]
                    return c2

                lax.fori_loop(0, hl, crow, 0)
            pltpu.sync_copy(rows_v, acc_sh.at[idx_v], add=True)
            return carry

        lax.fori_loop(0, _PW // _SCC, body, 0)
        plsc.subcore_barrier()

        @pl.when(sid < 10)
        def _():
            pltpu.sync_copy(acc_sh.at[pl.ds(sid * 1000, 1000)],
                            out_hbm.at[cid, pl.ds(sid * 1000, 1000)])

    return k(he, dst, zeros)


def _gru_edge(g, he, W3Whh, mb, Wih, bih, bhh):
    hw = he @ W3Whh
    m = jnp.maximum(g + hw[:, :H] + mb, 0.0)
    gi = m @ Wih + bih
    gh = hw[:, H:] + bhh
    r = jax.nn.sigmoid(gi[:, :H] + gh[:, :H])
    z = jax.nn.sigmoid(gi[:, H:2 * H] + gh[:, H:2 * H])
    n = jnp.tanh(gi[:, 2 * H:] + r * gh[:, 2 * H:])
    return (1.0 - z) * n + z * he


def _edge_body_first(g, efT, eW, eb, W3Whh, mb, Wih, bih, bhh, he_out):
    he = lax.dot_general(efT[...], eW[...], (((0,), (0,)), ((), ()))) + eb[...]
    he_out[...] = _gru_edge(g[...], he, W3Whh[...], mb[...],
                            Wih[...], bih[...], bhh[...])


def _edge_body_mid(g, he_in, W3Whh, mb, Wih, bih, bhh, he_out):
    he_out[...] = _gru_edge(g[...], he_in[...], W3Whh[...], mb[...],
                            Wih[...], bih[...], bhh[...])


def _edge_body_last(g, he_in, W3Whh, mb, Wih, bih, bhh, dW, db,
                    he_out, uefT_out):
    he = _gru_edge(g[...], he_in[...], W3Whh[...], mb[...],
                   Wih[...], bih[...], bhh[...])
    he_out[...] = he
    uefT_out[...] = lax.dot_general(dW[...], he,
                                    (((0,), (1,)), ((), ()))) + db[...]


def _eb(shape):
    return pl.BlockSpec(shape, lambda i: (i, 0))


def _wb(shape):
    return pl.BlockSpec(shape, lambda i: (0, 0))


def _edge_call(variant, args):
    grid = (_E2 // _BE,)
    f32 = jnp.float32
    if variant == 0:
        in_specs = [_eb((_BE, H)),
                    pl.BlockSpec((16, _BE), lambda i: (0, i)),
                    _wb((16, H)), _wb((1, H)), _wb((H, 4 * H)), _wb((1, H)),
                    _wb((H, 3 * H)), _wb((1, 3 * H)), _wb((1, 3 * H))]
        out_specs = _eb((_BE, H))
        out_shape = jax.ShapeDtypeStruct((_E2, H), f32)
        body = _edge_body_first
    elif variant == 1:
        in_specs = [_eb((_BE, H)), _eb((_BE, H)),
                    _wb((H, 4 * H)), _wb((1, H)),
                    _wb((H, 3 * H)), _wb((1, 3 * H)), _wb((1, 3 * H))]
        out_specs = _eb((_BE, H))
        out_shape = jax.ShapeDtypeStruct((_E2, H), f32)
        body = _edge_body_mid
    else:
        in_specs = [_eb((_BE, H)), _eb((_BE, H)),
                    _wb((H, 4 * H)), _wb((1, H)),
                    _wb((H, 3 * H)), _wb((1, 3 * H)), _wb((1, 3 * H)),
                    _wb((H, 16)), _wb((16, 1))]
        out_specs = [_eb((_BE, H)),
                     pl.BlockSpec((16, _BE), lambda i: (0, i))]
        out_shape = [jax.ShapeDtypeStruct((_E2, H), f32),
                     jax.ShapeDtypeStruct((16, _E2), f32)]
        body = _edge_body_last
    return pl.pallas_call(body, grid=grid, in_specs=in_specs,
                          out_specs=out_specs, out_shape=out_shape)(*args)


def _prep_body(nf, encW, encb, W12, hn_out, p12_out):
    hn = nf[...] @ encW[...] + encb[...]
    hn_out[...] = hn
    p12_out[...] = hn @ W12[...]


def _prep_call(nf, encW, encb, W12):
    f32 = jnp.float32
    return pl.pallas_call(
        _prep_body, grid=(N // _BN,),
        in_specs=[_eb((_BN, 128)), _wb((128, H)), _wb((1, H)), _wb((H, 2 * H))],
        out_specs=[_eb((_BN, H)), _eb((_BN, 2 * H))],
        out_shape=[jax.ShapeDtypeStruct((N, H), f32),
                   jax.ShapeDtypeStruct((N, 2 * H), f32)],
    )(nf, encW, encb, W12)


def _gru_node(agg, hn, Wih, bih, Whh, bhh):
    gi = agg @ Wih + bih
    gh = hn @ Whh + bhh
    r = jax.nn.sigmoid(gi[:, :H] + gh[:, :H])
    z = jax.nn.sigmoid(gi[:, H:2 * H] + gh[:, H:2 * H])
    n = jnp.tanh(gi[:, 2 * H:] + r * gh[:, 2 * H:])
    return (1.0 - z) * n + z * hn


def _node_body_mid(aggpA, aggpB, hn_in, Wih, bih, Whh, bhh, W12,
                   hn_out, p12_out):
    agg = (jnp.sum(aggpA[...], axis=0) + jnp.sum(aggpB[...], axis=0))[:, :H]
    hn = _gru_node(agg, hn_in[...], Wih[...], bih[...], Whh[...], bhh[...])
    hn_out[...] = hn
    p12_out[...] = hn @ W12[...]


def _node_body_last(aggpA, aggpB, hn_in, Wih, bih, Whh, bhh, dW, db, unf_out):
    agg = (jnp.sum(aggpA[...], axis=0) + jnp.sum(aggpB[...], axis=0))[:, :H]
    hn = _gru_node(agg, hn_in[...], Wih[...], bih[...], Whh[...], bhh[...])
    unf_out[...] = hn @ dW[...] + db[...]


def _node_call(variant, aggpA, aggpB, args):
    f32 = jnp.float32
    aspec = pl.BlockSpec((_NC, _BN, 2 * H), lambda i: (0, i, 0))
    if variant == 0:
        in_specs = [aspec, aspec, _eb((_BN, H)),
                    _wb((H, 3 * H)), _wb((1, 3 * H)), _wb((H, 3 * H)),
                    _wb((1, 3 * H)), _wb((H, 2 * H))]
        out_specs = [_eb((_BN, H)), _eb((_BN, 2 * H))]
        out_shape = [jax.ShapeDtypeStruct((N, H), f32),
                     jax.ShapeDtypeStruct((N, 2 * H), f32)]
        body = _node_body_mid
    else:
        in_specs = [aspec, aspec, _eb((_BN, H)),
                    _wb((H, 3 * H)), _wb((1, 3 * H)), _wb((H, 3 * H)),
                    _wb((1, 3 * H)), _wb((H, 128)), _wb((1, 128))]
        out_specs = _eb((_BN, 128))
        out_shape = jax.ShapeDtypeStruct((N, 128), f32)
        body = _node_body_last
    return pl.pallas_call(body, grid=(N // _BN,), in_specs=in_specs,
                          out_specs=out_specs, out_shape=out_shape)(aggpA, aggpB,
                                                                    *args)


def kernel(nf, ef, edge_index, node_enc_W, node_enc_b, edge_enc_W, edge_enc_b,
           msg_W, msg_b, e_gru_Wih, e_gru_Whh, e_gru_bih, e_gru_bhh,
           n_gru_Wih, n_gru_Whh, n_gru_bih, n_gru_bhh,
           node_dec_W, node_dec_b, edge_dec_W, edge_dec_b):
    src = edge_index[0]
    dst = edge_index[1]
    W12 = jnp.concatenate([msg_W[:H], msg_W[H:2 * H]], axis=1)
    W3Whh = jnp.concatenate([msg_W[2 * H:], e_gru_Whh], axis=1)
    mb = msg_b.reshape(1, H)
    ebih = e_gru_bih.reshape(1, 3 * H)
    ebhh = e_gru_bhh.reshape(1, 3 * H)
    nbih = n_gru_bih.reshape(1, 3 * H)
    nbhh = n_gru_bhh.reshape(1, 3 * H)

    hn, p12 = _prep_call(nf, node_enc_W, node_enc_b.reshape(1, H), W12)
    zeros = jnp.zeros((N, 2 * H), jnp.float32)

    srcs = (src[:_E2], src[_E2:])
    dsts = (dst[:_E2], dst[_E2:])
    efTs = (ef[:_E2].T, ef[_E2:].T)
    ebias = edge_enc_b.reshape(1, H)
    dbias = edge_dec_b.reshape(16, 1)

    he = [None, None]
    uef = [None, None]
    unf = None
    for it in range(3):
        aggp = [None, None]
        for hf in range(2):
            g = _gather_sc(p12, srcs[hf], dsts[hf])
            if it == 0:
                he[hf] = _edge_call(0, (g, efTs[hf], edge_enc_W, ebias,
                                        W3Whh, mb, e_gru_Wih, ebih, ebhh))
            elif it == 1:
                he[hf] = _edge_call(1, (g, he[hf], W3Whh, mb, e_gru_Wih,
                                        ebih, ebhh))
            else:
                he[hf], uef[hf] = _edge_call(2, (g, he[hf], W3Whh, mb,
                                                 e_gru_Wih, ebih, ebhh,
                                                 edge_dec_W, dbias))
            aggp[hf] = _scatter_sc(he[hf], dsts[hf], zeros)
        if it < 2:
            hn, p12 = _node_call(0, aggp[0], aggp[1],
                                 (hn, n_gru_Wih, nbih, n_gru_Whh, nbhh, W12))
        else:
            unf = _node_call(1, aggp[0], aggp[1],
                             (hn, n_gru_Wih, nbih, n_gru_Whh, nbhh,
                              node_dec_W, node_dec_b.reshape(1, 128)))
    return (unf, jnp.concatenate(uef, axis=1).T)
